# baseline (speedup 1.0000x reference)
import functools

import jax
import jax.numpy as jnp
from jax import lax
from jax.experimental import pallas as pl
from jax.experimental.pallas import tpu as pltpu
from jax.experimental.pallas import tpu_sc as plsc

CB = 1024
D = 32000
FB = 3200
NF = D // FB
K = 5
TEMP = 0.1
S = 2
DC = D // S
NW = 32


def _dist_kernel(l_ref, c_ref, idx_ref, w_ref, dist_ref, acc_ref):
    f = pl.program_id(0)
    l_blk = l_ref[...]
    c_blk = c_ref[...]
    dot = lax.dot_general(l_blk, c_blk, (((1,), (1,)), ((), ())),
                          preferred_element_type=jnp.float32)
    csq = c_blk * c_blk
    ones = jnp.ones((1, FB), jnp.float32)
    c2 = lax.dot_general(ones, csq, (((1,), (1,)), ((), ())),
                         preferred_element_type=jnp.float32)
    l2 = jnp.sum(l_blk * l_blk, axis=1, keepdims=True)
    part = l2 + c2 - 2.0 * dot

    @pl.when(f == 0)
    def _():
        acc_ref[...] = part

    @pl.when(f > 0)
    def _():
        acc_ref[...] = acc_ref[...] + part

    @pl.when(f == NF - 1)
    def _():
        d2 = acc_ref[...]
        lane = lax.broadcasted_iota(jnp.int32, (16, CB), 1)
        out_lane = lax.broadcasted_iota(jnp.int32, (16, 128), 1)
        idx_acc = jnp.zeros((16, 128), jnp.int32)
        d2_acc = jnp.zeros((16, 128), jnp.float32)
        cur = d2
        for k in range(K):
            mv = jnp.min(cur, axis=1, keepdims=True)
            cand = jnp.where(cur == mv, lane, CB)
            mi = jnp.min(cand, axis=1, keepdims=True)
            idx_acc = jnp.where(out_lane == k, mi, idx_acc)
            d2_acc = jnp.where(out_lane == k, mv, d2_acc)
            cur = jnp.where(lane == mi, jnp.float32(3.0e38), cur)
        dist = jnp.sqrt(jnp.maximum(d2_acc, 1e-12))
        valid = out_lane < K
        logits = -dist / TEMP
        m = jnp.max(jnp.where(valid, logits, -3.0e38), axis=1, keepdims=True)
        e = jnp.where(valid, jnp.exp(logits - m), 0.0)
        w = e / jnp.sum(e, axis=1, keepdims=True)
        idx_ref[...] = idx_acc
        w_ref[...] = w
        dist_ref[...] = dist


def _distances_top5(latent_flat, codebook_flat):
    out_shapes = (
        jax.ShapeDtypeStruct((16, 128), jnp.int32),
        jax.ShapeDtypeStruct((16, 128), jnp.float32),
        jax.ShapeDtypeStruct((16, 128), jnp.float32),
    )
    return pl.pallas_call(
        _dist_kernel,
        grid=(NF,),
        in_specs=[
            pl.BlockSpec((16, FB), lambda f: (0, f)),
            pl.BlockSpec((CB, FB), lambda f: (0, f)),
        ],
        out_specs=(
            pl.BlockSpec((16, 128), lambda f: (0, 0)),
            pl.BlockSpec((16, 128), lambda f: (0, 0)),
            pl.BlockSpec((16, 128), lambda f: (0, 0)),
        ),
        out_shape=out_shapes,
        scratch_shapes=[pltpu.VMEM((16, CB), jnp.float32)],
        compiler_params=pltpu.CompilerParams(
            dimension_semantics=("arbitrary",),
        ),
    )(latent_flat, codebook_flat)


RP = D // 128
HALF = RP // 2


def _sc_gather_body(table, gidx, wbc, out, gi_v, w_v, rows_v, acc_v, sem):
    w = lax.axis_index("c") * 16 + lax.axis_index("s")
    pltpu.sync_copy(gidx.at[w], gi_v)
    pltpu.sync_copy(wbc.at[w], w_v)
    copies = [
        pltpu.async_copy(table.at[gi_v.at[k]], rows_v.at[k], sem)
        for k in range(K)
    ]
    for cp in copies:
        cp.wait()
    wv = [w_v[k] for k in range(K)]

    def body(r, carry):
        for u in range(8):
            c = u * 16
            a = rows_v[0, r, pl.ds(c, 16)] * wv[0]
            a = a + rows_v[1, r, pl.ds(c, 16)] * wv[1]
            a = a + rows_v[2, r, pl.ds(c, 16)] * wv[2]
            a = a + rows_v[3, r, pl.ds(c, 16)] * wv[3]
            a = a + rows_v[4, r, pl.ds(c, 16)] * wv[4]
            acc_v[r, pl.ds(c, 16)] = a
        return carry

    lax.fori_loop(0, HALF, body, 0)
    pltpu.sync_copy(acc_v, out.at[pl.ds(w * 128, 128)])


@functools.lru_cache(maxsize=1)
def _sc_gather_kernel():
    mesh = plsc.VectorSubcoreMesh(core_axis_name="c", subcore_axis_name="s")
    return pl.kernel(
        _sc_gather_body,
        out_type=jax.ShapeDtypeStruct((NW * 128, 128), jnp.float32),
        mesh=mesh,
        scratch_types=[
            pltpu.VMEM((K, HALF), jnp.int32),
            pltpu.VMEM((K, 16), jnp.float32),
            pltpu.VMEM((K, HALF, 128), jnp.float32),
            pltpu.VMEM((128, 128), jnp.float32),
            pltpu.SemaphoreType.DMA,
        ],
    )


def kernel(latent, codebook):
    B = latent.shape[0]
    latent_flat = latent.reshape(B, -1).astype(jnp.float32)
    codebook_flat = codebook.reshape(CB, -1).astype(jnp.float32)

    idx_pad, w_pad, dist_pad = _distances_top5(latent_flat, codebook_flat)

    idx5 = idx_pad[:, :K]
    main_indices = idx_pad[:, 0]
    main_distances = dist_pad[:, 0]

    idx5r = jnp.repeat(idx5, 2, axis=0)
    half_off = (jnp.arange(NW, dtype=jnp.int32) % 2) * HALF
    gidx = (idx5r * RP + half_off[:, None])[:, :, None] \
        + jnp.arange(HALF, dtype=jnp.int32)[None, None, :]
    wbc = jnp.repeat(w_pad[:, :K], 2, axis=0)
    wbc = jnp.broadcast_to(wbc[:, :, None], (NW, K, 16))

    table = codebook_flat.reshape(CB * RP, 128)
    quant = _sc_gather_kernel()(table, gidx, wbc.astype(jnp.float32))
    quant = quant.reshape(NW, 128, 128)[:, :HALF, :]
    quantized = quant.reshape(latent.shape).astype(latent.dtype)
    return quantized, main_indices, main_distances


# speedup vs baseline: 2.9413x; 2.9413x over previous
import functools

import jax
import jax.numpy as jnp
from jax import lax
from jax.experimental import pallas as pl
from jax.experimental.pallas import tpu as pltpu
from jax.experimental.pallas import tpu_sc as plsc

CB = 1024
D = 32000
RB = 128
NRB = CB // RB
K = 5
TEMP = 0.1
NW = 32
DPAD = 32768
SRP = DPAD // 128
HB = 128


def _dist_kernel(l_ref, c_ref, idx_ref, w_ref, dist_ref, acc_ref):
    j = pl.program_id(0)
    l_blk = l_ref[...]
    c_blk = c_ref[...]
    dot = lax.dot_general(l_blk, c_blk, (((1,), (1,)), ((), ())),
                          preferred_element_type=jnp.float32)
    csq = c_blk * c_blk
    ones = jnp.ones((1, D), jnp.float32)
    c2 = lax.dot_general(ones, csq, (((1,), (1,)), ((), ())),
                         preferred_element_type=jnp.float32)
    l2 = jnp.sum(l_blk * l_blk, axis=1, keepdims=True)
    acc_ref[:, pl.ds(j * RB, RB)] = l2 + c2 - 2.0 * dot

    @pl.when(j == NRB - 1)
    def _():
        d2 = acc_ref[...]
        lane = lax.broadcasted_iota(jnp.int32, (16, CB), 1)
        out_lane = lax.broadcasted_iota(jnp.int32, (16, 128), 1)
        idx_acc = jnp.zeros((16, 128), jnp.int32)
        d2_acc = jnp.zeros((16, 128), jnp.float32)
        cur = d2
        for k in range(K):
            mv = jnp.min(cur, axis=1, keepdims=True)
            cand = jnp.where(cur == mv, lane, CB)
            mi = jnp.min(cand, axis=1, keepdims=True)
            idx_acc = jnp.where(out_lane == k, mi, idx_acc)
            d2_acc = jnp.where(out_lane == k, mv, d2_acc)
            cur = jnp.where(lane == mi, jnp.float32(3.0e38), cur)
        dist = jnp.sqrt(jnp.maximum(d2_acc, 1e-12))
        valid = out_lane < K
        logits = -dist / TEMP
        m = jnp.max(jnp.where(valid, logits, -3.0e38), axis=1, keepdims=True)
        e = jnp.where(valid, jnp.exp(logits - m), 0.0)
        w = e / jnp.sum(e, axis=1, keepdims=True)
        idx_ref[...] = idx_acc
        w_ref[...] = w
        dist_ref[...] = dist


def _distances_top5(latent_flat, codebook_flat):
    out_shapes = (
        jax.ShapeDtypeStruct((16, 128), jnp.int32),
        jax.ShapeDtypeStruct((16, 128), jnp.float32),
        jax.ShapeDtypeStruct((16, 128), jnp.float32),
    )
    return pl.pallas_call(
        _dist_kernel,
        grid=(NRB,),
        in_specs=[
            pl.BlockSpec((16, D), lambda j: (0, 0)),
            pl.BlockSpec((RB, D), lambda j: (j, 0)),
        ],
        out_specs=(
            pl.BlockSpec((16, 128), lambda j: (0, 0)),
            pl.BlockSpec((16, 128), lambda j: (0, 0)),
            pl.BlockSpec((16, 128), lambda j: (0, 0)),
        ),
        out_shape=out_shapes,
        scratch_shapes=[pltpu.VMEM((16, CB), jnp.float32)],
        compiler_params=pltpu.CompilerParams(
            dimension_semantics=("arbitrary",),
        ),
    )(latent_flat, codebook_flat)


def _gather_kernel(idx_ref, c_hbm, out_ref, sem):
    copies = []
    for i in range(16 * K):
        cp = pltpu.make_async_copy(
            c_hbm.at[pl.ds(idx_ref[i], 1)],
            out_ref.at[pl.ds(i, 1), pl.ds(0, D)],
            sem,
        )
        cp.start()
        copies.append(cp)
    out_ref[:, pl.ds(D, DPAD - D)] = jnp.zeros((16 * K, DPAD - D), jnp.float32)
    for cp in copies:
        cp.wait()


def _gather_rows(codebook_flat, idx_flat):
    grid_spec = pltpu.PrefetchScalarGridSpec(
        num_scalar_prefetch=1,
        grid=(1,),
        in_specs=[pl.BlockSpec(memory_space=pltpu.MemorySpace.HBM)],
        out_specs=pl.BlockSpec((16 * K, DPAD), lambda i, idx: (0, 0)),
        scratch_shapes=[pltpu.SemaphoreType.DMA],
    )
    return pl.pallas_call(
        _gather_kernel,
        grid_spec=grid_spec,
        out_shape=jax.ShapeDtypeStruct((16 * K, DPAD), jnp.float32),
    )(idx_flat, codebook_flat)


def _sc_wsum_body(table, wbc, out, w_v, rows_v, acc_v, sem):
    w = lax.axis_index("c") * 16 + lax.axis_index("s")
    b = w // 2
    h = w % 2
    pltpu.sync_copy(wbc.at[w], w_v)
    copies = [
        pltpu.async_copy(
            table.at[pl.ds(pl.multiple_of((b * K + k) * SRP + h * HB, 8), HB)],
            rows_v.at[k], sem)
        for k in range(K)
    ]
    for cp in copies:
        cp.wait()
    wv = [w_v[k] for k in range(K)]

    def body(r, carry):
        for u in range(8):
            c = u * 16
            a = rows_v[0, r, pl.ds(c, 16)] * wv[0]
            a = a + rows_v[1, r, pl.ds(c, 16)] * wv[1]
            a = a + rows_v[2, r, pl.ds(c, 16)] * wv[2]
            a = a + rows_v[3, r, pl.ds(c, 16)] * wv[3]
            a = a + rows_v[4, r, pl.ds(c, 16)] * wv[4]
            acc_v[r, pl.ds(c, 16)] = a
        return carry

    lax.fori_loop(0, HB, body, 0)
    pltpu.sync_copy(acc_v, out.at[pl.ds(w * HB, HB)])


@functools.lru_cache(maxsize=1)
def _sc_wsum_kernel():
    mesh = plsc.VectorSubcoreMesh(core_axis_name="c", subcore_axis_name="s")
    return pl.kernel(
        _sc_wsum_body,
        out_type=jax.ShapeDtypeStruct((NW * HB, 128), jnp.float32),
        mesh=mesh,
        scratch_types=[
            pltpu.VMEM((K, 16), jnp.float32),
            pltpu.VMEM((K, HB, 128), jnp.float32),
            pltpu.VMEM((HB, 128), jnp.float32),
            pltpu.SemaphoreType.DMA,
        ],
    )


def kernel(latent, codebook):
    B = latent.shape[0]
    latent_flat = latent.reshape(B, -1).astype(jnp.float32)
    codebook_flat = codebook.reshape(CB, -1).astype(jnp.float32)

    idx_pad, w_pad, dist_pad = _distances_top5(latent_flat, codebook_flat)

    idx5 = idx_pad[:, :K]
    main_indices = idx_pad[:, 0]
    main_distances = dist_pad[:, 0]

    g = _gather_rows(codebook_flat, idx5.reshape(16 * K))
    table = g.reshape(16 * K * SRP, 128)

    wbc = jnp.repeat(w_pad[:, :K], 2, axis=0)
    wbc = jnp.broadcast_to(wbc[:, :, None], (NW, K, 16))

    out = _sc_wsum_kernel()(table, wbc.astype(jnp.float32))
    qa = out.reshape(16, 2, HB, 128)
    quant = jnp.concatenate([qa[:, 0], qa[:, 1, : 250 - HB]], axis=1)
    quantized = quant.reshape(latent.shape).astype(latent.dtype)
    return quantized, main_indices, main_distances


# speedup vs baseline: 5.3640x; 1.8237x over previous
import jax
import jax.numpy as jnp
from jax import lax
from jax.experimental import pallas as pl
from jax.experimental.pallas import tpu as pltpu

CB = 1024
D = 32000
KB = 3200
NKB = D // KB
K = 5
TEMP = 0.1


def _dist_kernel(l_ref, ct_ref, idx_ref, w_ref, dist_ref, wsp_ref, acc_ref):
    k_step = pl.program_id(0)
    l_blk = l_ref[...]
    ct_blk = ct_ref[...]
    dot = lax.dot_general(l_blk, ct_blk, (((1,), (0,)), ((), ())),
                          preferred_element_type=jnp.float32)
    c2 = jnp.sum(ct_blk * ct_blk, axis=0, keepdims=True)
    l2 = jnp.sum(l_blk * l_blk, axis=1, keepdims=True)
    part = l2 + c2 - 2.0 * dot

    @pl.when(k_step == 0)
    def _():
        acc_ref[...] = part

    @pl.when(k_step > 0)
    def _():
        acc_ref[...] = acc_ref[...] + part

    @pl.when(k_step == NKB - 1)
    def _():
        d2 = acc_ref[...]
        lane = lax.broadcasted_iota(jnp.int32, (16, CB), 1)
        out_lane = lax.broadcasted_iota(jnp.int32, (16, 128), 1)
        idx_acc = jnp.zeros((16, 128), jnp.int32)
        d2_acc = jnp.zeros((16, 128), jnp.float32)
        cur = d2
        mis = []
        for k in range(K):
            mv = jnp.min(cur, axis=1, keepdims=True)
            cand = jnp.where(cur == mv, lane, CB)
            mi = jnp.min(cand, axis=1, keepdims=True)
            mis.append(mi)
            idx_acc = jnp.where(out_lane == k, mi, idx_acc)
            d2_acc = jnp.where(out_lane == k, mv, d2_acc)
            cur = jnp.where(lane == mi, jnp.float32(3.0e38), cur)
        dist = jnp.sqrt(jnp.maximum(d2_acc, 1e-12))
        valid = out_lane < K
        logits = -dist / TEMP
        m = jnp.max(jnp.where(valid, logits, -3.0e38), axis=1, keepdims=True)
        e = jnp.where(valid, jnp.exp(logits - m), 0.0)
        w = e / jnp.sum(e, axis=1, keepdims=True)
        wsp = jnp.zeros((16, CB), jnp.float32)
        for k in range(K):
            wk = jnp.sum(jnp.where(out_lane == k, w, 0.0), axis=1,
                         keepdims=True)
            wsp = jnp.where(lane == mis[k], wk, wsp)
        idx_ref[...] = idx_acc
        w_ref[...] = w
        dist_ref[...] = dist
        wsp_ref[...] = wsp


def _distances_top5(latent_flat, ct):
    out_shapes = (
        jax.ShapeDtypeStruct((16, 128), jnp.int32),
        jax.ShapeDtypeStruct((16, 128), jnp.float32),
        jax.ShapeDtypeStruct((16, 128), jnp.float32),
        jax.ShapeDtypeStruct((16, CB), jnp.float32),
    )
    return pl.pallas_call(
        _dist_kernel,
        grid=(NKB,),
        in_specs=[
            pl.BlockSpec((16, KB), lambda k: (0, k)),
            pl.BlockSpec((KB, CB), lambda k: (k, 0)),
        ],
        out_specs=(
            pl.BlockSpec((16, 128), lambda k: (0, 0)),
            pl.BlockSpec((16, 128), lambda k: (0, 0)),
            pl.BlockSpec((16, 128), lambda k: (0, 0)),
            pl.BlockSpec((16, CB), lambda k: (0, 0)),
        ),
        out_shape=out_shapes,
        scratch_shapes=[pltpu.VMEM((16, CB), jnp.float32)],
        compiler_params=pltpu.CompilerParams(
            dimension_semantics=("arbitrary",),
        ),
    )(latent_flat, ct)


def _wsum_kernel(wsp_ref, ct_ref, out_ref):
    out_ref[...] = lax.dot_general(
        wsp_ref[...], ct_ref[...], (((1,), (1,)), ((), ())),
        preferred_element_type=jnp.float32)


def _weighted_sum(wsp, ct):
    return pl.pallas_call(
        _wsum_kernel,
        grid=(NKB,),
        in_specs=[
            pl.BlockSpec((16, CB), lambda k: (0, 0)),
            pl.BlockSpec((KB, CB), lambda k: (k, 0)),
        ],
        out_specs=pl.BlockSpec((16, KB), lambda k: (0, k)),
        out_shape=jax.ShapeDtypeStruct((16, D), jnp.float32),
        compiler_params=pltpu.CompilerParams(
            dimension_semantics=("arbitrary",),
        ),
    )(wsp, ct)


def kernel(latent, codebook):
    B = latent.shape[0]
    latent_flat = latent.reshape(B, -1).astype(jnp.float32)
    ct = codebook.transpose(1, 2, 3, 0).reshape(D, CB).astype(jnp.float32)

    idx_pad, w_pad, dist_pad, wsp = _distances_top5(latent_flat, ct)
    main_indices = idx_pad[:, 0]
    main_distances = dist_pad[:, 0]

    quantized_flat = _weighted_sum(wsp, ct)
    quantized = quantized_flat.reshape(latent.shape).astype(latent.dtype)
    return quantized, main_indices, main_distances
